# Initial kernel scaffold; baseline (speedup 1.0000x reference)
import jax
import jax.numpy as jnp
from jax.experimental import pallas as pl


def kernel(x_f0, x_org, c_trg, params):
    raise NotImplementedError("write your pallas kernel here")


# speedup vs baseline: 4.0150x; 4.0150x over previous
import functools

import jax
import jax.numpy as jnp
from jax.experimental import pallas as pl
from jax.experimental.pallas import tpu as pltpu


def _conv_kern(x_ref, w_ref, b_ref, o_ref, *, K, T, gn_groups, g_ref=None,
               bt_ref=None):
    acc = jnp.zeros((T, w_ref.shape[2]), jnp.float32)
    for k in range(K):
        acc += jax.lax.dot_general(
            x_ref[0, k:k + T, :], w_ref[k],
            (((1,), (0,)), ((), ())), preferred_element_type=jnp.float32)
    acc = jnp.maximum(acc + b_ref[0][None, :], 0.0)
    if gn_groups:
        cs = acc.shape[1] // gn_groups
        gam = g_ref[0][None, :]
        bet = bt_ref[0][None, :]
        cols = []
        for g in range(gn_groups):
            sl = acc[:, g * cs:(g + 1) * cs]
            mu = jnp.mean(sl)
            var = jnp.mean((sl - mu) * (sl - mu))
            cols.append((sl - mu) * jax.lax.rsqrt(var + 1e-5))
        acc = jnp.concatenate(cols, axis=1)
        acc = jnp.maximum(acc * gam + bet, 0.0)
    o_ref[0] = acc


def _conv1d(x, W, b, gn=None, gn_groups=0):
    B, T, cin = x.shape
    cout, _, K = W.shape
    pad = (K - 1) // 2
    xp = jnp.pad(x, ((0, 0), (pad, pad), (0, 0)))
    wt = jnp.transpose(W, (2, 1, 0))
    b2 = b.reshape(1, cout)
    ins = [xp, wt, b2]
    in_specs = [
        pl.BlockSpec((1, T + K - 1, cin), lambda i: (i, 0, 0)),
        pl.BlockSpec((K, cin, cout), lambda i: (0, 0, 0)),
        pl.BlockSpec((1, cout), lambda i: (0, 0)),
    ]
    if gn_groups:
        ins += [gn['g'].reshape(1, cout), gn['b'].reshape(1, cout)]
        in_specs += [pl.BlockSpec((1, cout), lambda i: (0, 0))] * 2
        kern = functools.partial(_conv_kern, K=K, T=T, gn_groups=gn_groups)

        def wrapped(x_r, w_r, b_r, g_r, bt_r, o_r):
            kern(x_r, w_r, b_r, o_r, g_ref=g_r, bt_ref=bt_r)
        body = wrapped
    else:
        body = functools.partial(_conv_kern, K=K, T=T, gn_groups=0)
    return pl.pallas_call(
        body,
        grid=(B,),
        in_specs=in_specs,
        out_specs=pl.BlockSpec((1, T, cout), lambda i: (i, 0, 0)),
        out_shape=jax.ShapeDtypeStruct((B, T, cout), jnp.float32),
    )(*ins)


def _mm_kern(a_ref, w_ref, b_ref, o_ref):
    o_ref[...] = jax.lax.dot_general(
        a_ref[...], w_ref[...], (((1,), (0,)), ((), ())),
        preferred_element_type=jnp.float32) + b_ref[0][None, :]


def _matmul(a, w, bias):
    M, K = a.shape
    N = w.shape[1]
    bm = M if M <= 768 else 768
    assert M % bm == 0
    bn = N if N <= 2048 else 2048
    assert N % bn == 0
    return pl.pallas_call(
        _mm_kern,
        grid=(N // bn, M // bm),
        in_specs=[
            pl.BlockSpec((bm, K), lambda j, i: (i, 0)),
            pl.BlockSpec((K, bn), lambda j, i: (0, j)),
            pl.BlockSpec((1, bn), lambda j, i: (0, j)),
        ],
        out_specs=pl.BlockSpec((bm, bn), lambda j, i: (i, j)),
        out_shape=jax.ShapeDtypeStruct((M, N), jnp.float32),
    )(a, w, bias.reshape(1, N))


def _cell(z, h, c, H):
    i = jax.nn.sigmoid(z[:, :H])
    f = jax.nn.sigmoid(z[:, H:2 * H])
    g = jnp.tanh(z[:, 2 * H:3 * H])
    o = jax.nn.sigmoid(z[:, 3 * H:])
    c = f * c + i * g
    h = o * jnp.tanh(c)
    return h, c


def _rec(h, w_ref, H):
    if H == 1:
        return h * w_ref[...]
    return jax.lax.dot_general(h, w_ref[...], (((1,), (0,)), ((), ())),
                               preferred_element_type=jnp.float32)


def _bilstm_small_kern(xf_ref, xb_ref, wf_ref, wb_ref, yf_ref, yb_ref, *,
                       T, H):
    B = xf_ref.shape[0]
    zero = jnp.zeros((B, H), jnp.float32)

    def body(t, carry):
        hf, cf, hb, cb = carry
        zf = xf_ref[:, t, :] + _rec(hf, wf_ref, H)
        hf, cf = _cell(zf, hf, cf, H)
        yf_ref[:, pl.ds(t, 1), :] = hf[:, None, :]
        tb = T - 1 - t
        zb = xb_ref[:, tb, :] + _rec(hb, wb_ref, H)
        hb, cb = _cell(zb, hb, cb, H)
        yb_ref[:, pl.ds(tb, 1), :] = hb[:, None, :]
        return hf, cf, hb, cb

    jax.lax.fori_loop(0, T, body, (zero, zero, zero, zero))


def _bilstm_big_kern(xf_ref, xb_ref, wf_ref, wb_ref, yf_ref, yb_ref,
                     hf_s, cf_s, hb_s, cb_s, *, TC, H):
    @pl.when(pl.program_id(0) == 0)
    def _init():
        hf_s[...] = jnp.zeros_like(hf_s)
        cf_s[...] = jnp.zeros_like(cf_s)
        hb_s[...] = jnp.zeros_like(hb_s)
        cb_s[...] = jnp.zeros_like(cb_s)

    def body(t, _):
        hf, cf = _cell(xf_ref[:, t, :] + _rec(hf_s[...], wf_ref, H),
                       hf_s[...], cf_s[...], H)
        yf_ref[:, pl.ds(t, 1), :] = hf[:, None, :]
        hf_s[...] = hf
        cf_s[...] = cf
        tb = TC - 1 - t
        hb, cb = _cell(xb_ref[:, tb, :] + _rec(hb_s[...], wb_ref, H),
                       hb_s[...], cb_s[...], H)
        yb_ref[:, pl.ds(tb, 1), :] = hb[:, None, :]
        hb_s[...] = hb
        cb_s[...] = cb
        return 0

    jax.lax.fori_loop(0, TC, body, 0)


def _bilstm(x, p, big_chunks=0):
    B, T, I = x.shape
    Wih_f, Whh_f, bih_f, bhh_f = p['fwd']
    Wih_b, Whh_b, bih_b, bhh_b = p['bwd']
    H = Whh_f.shape[1]
    wcat = jnp.concatenate([Wih_f.T, Wih_b.T], axis=1)
    bcat = jnp.concatenate([bih_f + bhh_f, bih_b + bhh_b])
    xp = _matmul(x.reshape(B * T, I), wcat, bcat).reshape(B, T, 8 * H)
    xf = xp[:, :, :4 * H]
    xb = xp[:, :, 4 * H:]
    wf = Whh_f.T
    wb = Whh_b.T
    out_shape = [jax.ShapeDtypeStruct((B, T, H), jnp.float32)] * 2
    if big_chunks:
        NC = big_chunks
        TC = T // NC
        yf, yb = pl.pallas_call(
            functools.partial(_bilstm_big_kern, TC=TC, H=H),
            grid=(NC,),
            in_specs=[
                pl.BlockSpec((B, TC, 4 * H), lambda i: (0, i, 0)),
                pl.BlockSpec((B, TC, 4 * H), lambda i, NC=NC: (0, NC - 1 - i, 0)),
                pl.BlockSpec((H, 4 * H), lambda i: (0, 0)),
                pl.BlockSpec((H, 4 * H), lambda i: (0, 0)),
            ],
            out_specs=[
                pl.BlockSpec((B, TC, H), lambda i: (0, i, 0)),
                pl.BlockSpec((B, TC, H), lambda i, NC=NC: (0, NC - 1 - i, 0)),
            ],
            out_shape=out_shape,
            scratch_shapes=[pltpu.VMEM((B, H), jnp.float32)] * 4,
            compiler_params=pltpu.CompilerParams(
                dimension_semantics=("arbitrary",)),
        )(xf, xb, wf, wb)
    else:
        yf, yb = pl.pallas_call(
            functools.partial(_bilstm_small_kern, T=T, H=H),
            in_specs=[pl.BlockSpec(a.shape, functools.partial(
                          lambda n: (0,) * n, a.ndim))
                      for a in (xf, xb, wf, wb)],
            out_specs=[pl.BlockSpec((B, T, H), lambda: (0, 0, 0))] * 2,
            out_shape=out_shape,
        )(xf, xb, wf, wb)
    return jnp.concatenate([yf, yb], axis=-1)


def kernel(x_f0, x_org, c_trg, params):
    B, T, _ = x_f0.shape
    c = x_f0[:, :, :8]
    f = x_f0[:, :, 8:]
    for i in range(2):
        c = _conv1d(c, params['conv_c'][i]['W'], params['conv_c'][i]['b'])
        f = _conv1d(f, params['conv_f'][i]['W'], params['conv_f'][i]['b'])
    c = _conv1d(c, params['conv_c'][2]['W'], params['conv_c'][2]['b'],
                gn=params['gn_c'], gn_groups=32)
    f = _conv1d(f, params['conv_f'][2]['W'], params['conv_f'][2]['b'],
                gn=params['gn_f'], gn_groups=16)
    c_out = _bilstm(c, params['lstm_c'])
    f_out = _bilstm(f, params['lstm_f'])
    r = _conv1d(x_org, params['conv_r']['W'], params['conv_r']['b'],
                gn=params['gn_r'], gn_groups=8)
    r_out = _bilstm(r, params['lstm_r'])
    codes_c = jnp.concatenate([c_out[:, 7::8, :8], c_out[:, ::8, 8:]], axis=-1)
    codes_f = jnp.concatenate([f_out[:, 7::8, :32], f_out[:, ::8, 32:]], axis=-1)
    codes_r = jnp.concatenate([r_out[:, 7::8, :1], r_out[:, ::8, 1:]], axis=-1)
    enc = jnp.concatenate([
        jnp.repeat(codes_c, 8, axis=1),
        jnp.repeat(codes_r, 8, axis=1),
        jnp.repeat(codes_f, 8, axis=1),
        jnp.broadcast_to(c_trg[:, None, :], (B, T, c_trg.shape[-1])),
    ], axis=-1)
    h = enc
    for layer in params['lstm_d']:
        h = _bilstm(h, layer, big_chunks=12)
    lin = params['linear']
    out = _matmul(h.reshape(B * T, h.shape[-1]), lin['W'].T, lin['b'])
    return out.reshape(B, T, lin['W'].shape[0])


# speedup vs baseline: 4.6697x; 1.1631x over previous
import functools

import jax
import jax.numpy as jnp
import numpy as np
from jax.experimental import pallas as pl
from jax.experimental.pallas import tpu as pltpu


def _mmdot(a, b):
    return jax.lax.dot_general(a, b, (((1,), (0,)), ((), ())),
                               preferred_element_type=jnp.float32)


def _conv_kern(x_ref, w_ref, b_ref, o_ref, *, K, T, gn_groups, g_ref=None,
               bt_ref=None, mcg_ref=None, mgc_ref=None):
    acc = jnp.zeros((T, w_ref.shape[2]), jnp.float32)
    for k in range(K):
        acc += _mmdot(x_ref[0, k:k + T, :], w_ref[k])
    acc = jnp.maximum(acc + b_ref[0][None, :], 0.0)
    if gn_groups:
        C = acc.shape[1]
        cs = C // gn_groups
        n = float(cs * T)
        s1 = jnp.sum(acc, axis=0, keepdims=True)
        s2 = jnp.sum(acc * acc, axis=0, keepdims=True)
        g1 = _mmdot(s1, mcg_ref[...]) / n
        g2 = _mmdot(s2, mcg_ref[...]) / n
        rstd = jax.lax.rsqrt(g2 - g1 * g1 + 1e-5)
        mu_c = _mmdot(g1, mgc_ref[...])
        rs_c = _mmdot(rstd, mgc_ref[...])
        a = rs_c * g_ref[0][None, :]
        acc = jnp.maximum(acc * a + (bt_ref[0][None, :] - mu_c * a), 0.0)
    o_ref[0] = acc


def _conv1d(x, W, b, gn=None, gn_groups=0):
    B, T, cin = x.shape
    cout, _, K = W.shape
    pad = (K - 1) // 2
    xp = jnp.pad(x, ((0, 0), (pad, pad), (0, 0)))
    wt = jnp.transpose(W, (2, 1, 0))
    b2 = b.reshape(1, cout)
    ins = [xp, wt, b2]
    in_specs = [
        pl.BlockSpec((1, T + K - 1, cin), lambda i: (i, 0, 0)),
        pl.BlockSpec((K, cin, cout), lambda i: (0, 0, 0)),
        pl.BlockSpec((1, cout), lambda i: (0, 0)),
    ]
    if gn_groups:
        mcg = jnp.repeat(jnp.eye(gn_groups, dtype=jnp.float32),
                         cout // gn_groups, axis=0)
        ins += [gn['g'].reshape(1, cout), gn['b'].reshape(1, cout),
                mcg, mcg.T]
        in_specs += [pl.BlockSpec((1, cout), lambda i: (0, 0))] * 2 + [
            pl.BlockSpec((cout, gn_groups), lambda i: (0, 0)),
            pl.BlockSpec((gn_groups, cout), lambda i: (0, 0)),
        ]
        kern = functools.partial(_conv_kern, K=K, T=T, gn_groups=gn_groups)

        def wrapped(x_r, w_r, b_r, g_r, bt_r, mcg_r, mgc_r, o_r):
            kern(x_r, w_r, b_r, o_r, g_ref=g_r, bt_ref=bt_r,
                 mcg_ref=mcg_r, mgc_ref=mgc_r)
        body = wrapped
    else:
        body = functools.partial(_conv_kern, K=K, T=T, gn_groups=0)
    return pl.pallas_call(
        body,
        grid=(B,),
        in_specs=in_specs,
        out_specs=pl.BlockSpec((1, T, cout), lambda i: (i, 0, 0)),
        out_shape=jax.ShapeDtypeStruct((B, T, cout), jnp.float32),
    )(*ins)


def _mm_kern(a_ref, w_ref, b_ref, o_ref):
    o_ref[...] = jax.lax.dot_general(
        a_ref[...], w_ref[...], (((1,), (0,)), ((), ())),
        preferred_element_type=jnp.float32) + b_ref[0][None, :]


def _matmul(a, w, bias):
    M, K = a.shape
    N = w.shape[1]
    bm = M if M <= 768 else 768
    assert M % bm == 0
    bn = N if N <= 2048 else 2048
    assert N % bn == 0
    return pl.pallas_call(
        _mm_kern,
        grid=(N // bn, M // bm),
        in_specs=[
            pl.BlockSpec((bm, K), lambda j, i: (i, 0)),
            pl.BlockSpec((K, bn), lambda j, i: (0, j)),
            pl.BlockSpec((1, bn), lambda j, i: (0, j)),
        ],
        out_specs=pl.BlockSpec((bm, bn), lambda j, i: (i, j)),
        out_shape=jax.ShapeDtypeStruct((M, N), jnp.float32),
    )(a, w, bias.reshape(1, N))


_GSRC = (0, 1, 3, 2)


def _permg(w4, H):
    return jnp.concatenate([w4[..., q * H:(q + 1) * H] for q in _GSRC],
                           axis=-1)


def _cellp(z, c, H):
    s = jax.nn.sigmoid(z[:, :3 * H])
    g = jnp.tanh(z[:, 3 * H:])
    c = s[:, H:2 * H] * c + s[:, :H] * g
    h = s[:, 2 * H:3 * H] * jnp.tanh(c)
    return h, c


def _rec(h, w_ref):
    return jax.lax.dot_general(h, w_ref[...], (((1,), (0,)), ((), ())),
                               preferred_element_type=jnp.float32)


def _bilstm_pack_kern(xf_ref, xb_ref, wf_ref, wb_ref, yf_ref, yb_ref, *,
                      T, HT):
    B = xf_ref.shape[0]
    zero = jnp.zeros((B, HT), jnp.float32)

    def body(t, carry):
        hf, cf, hb, cb = carry
        tb = T - 1 - t
        hf, cf = _cellp(xf_ref[:, t, :] + _rec(hf, wf_ref), cf, HT)
        yf_ref[:, pl.ds(t, 1), :] = hf[:, None, :]
        hb, cb = _cellp(xb_ref[:, tb, :] + _rec(hb, wb_ref), cb, HT)
        yb_ref[:, pl.ds(tb, 1), :] = hb[:, None, :]
        return hf, cf, hb, cb

    jax.lax.fori_loop(0, T, body, (zero, zero, zero, zero))


def _bilstm_multi(xs, ps):
    B, T = xs[0].shape[:2]
    Hs = [int(p['fwd'][1].shape[1]) for p in ps]
    Is = [int(x.shape[-1]) for x in xs]
    HT, IT = sum(Hs), sum(Is)
    offH = np.concatenate([[0], np.cumsum(Hs)])
    offI = np.concatenate([[0], np.cumsum(Is)])
    Wall = jnp.zeros((IT, 8 * HT), jnp.float32)
    ball = jnp.zeros((8 * HT,), jnp.float32)
    Whh_d = [jnp.zeros((HT, 4 * HT), jnp.float32) for _ in range(2)]
    for k, p in enumerate(ps):
        H = Hs[k]
        for d, dirn in enumerate(('fwd', 'bwd')):
            Wih, Whh, bih, bhh = p[dirn]
            WT, HhT, bb = Wih.T, Whh.T, bih + bhh
            for q in range(4):
                sq = _GSRC[q]
                col = d * 4 * HT + q * HT + offH[k]
                Wall = Wall.at[offI[k]:offI[k + 1], col:col + H].set(
                    WT[:, sq * H:(sq + 1) * H])
                ball = ball.at[col:col + H].set(bb[sq * H:(sq + 1) * H])
                rcol = q * HT + offH[k]
                Whh_d[d] = Whh_d[d].at[offH[k]:offH[k + 1],
                                       rcol:rcol + H].set(
                    HhT[:, sq * H:(sq + 1) * H])
    xcat = jnp.concatenate(xs, axis=-1).reshape(B * T, IT)
    xp = _matmul(xcat, Wall, ball).reshape(B, T, 8 * HT)
    xf, xb = xp[..., :4 * HT], xp[..., 4 * HT:]
    ins = (xf, xb, Whh_d[0], Whh_d[1])
    yf, yb = pl.pallas_call(
        functools.partial(_bilstm_pack_kern, T=T, HT=HT),
        in_specs=[pl.BlockSpec(a.shape, functools.partial(
                      lambda nd: (0,) * nd, a.ndim)) for a in ins],
        out_specs=[pl.BlockSpec((B, T, HT), lambda: (0, 0, 0))] * 2,
        out_shape=[jax.ShapeDtypeStruct((B, T, HT), jnp.float32)] * 2,
    )(*ins)
    return [jnp.concatenate([yf[..., offH[k]:offH[k + 1]],
                             yb[..., offH[k]:offH[k + 1]]], axis=-1)
            for k in range(len(ps))]


def _lstm_proj(x, p):
    B, T, I = x.shape
    Wih_f, Whh_f, bih_f, bhh_f = p['fwd']
    Wih_b, Whh_b, bih_b, bhh_b = p['bwd']
    H = Whh_f.shape[1]
    wcat = jnp.concatenate([_permg(Wih_f.T, H), _permg(Wih_b.T, H)], axis=1)
    bcat = jnp.concatenate([_permg(bih_f + bhh_f, H),
                            _permg(bih_b + bhh_b, H)])
    xp = _matmul(x.reshape(B * T, I), wcat, bcat).reshape(B, T, 8 * H)
    return (xp[:, :, :4 * H], xp[:, :, 4 * H:],
            _permg(Whh_f.T, H), _permg(Whh_b.T, H))


def _bilstm_big_kern(xf_ref, xb_ref, wf_ref, wb_ref, yf_ref, yb_ref,
                     hf_s, cf_s, hb_s, cb_s, *, TC, H):
    @pl.when(pl.program_id(0) == 0)
    def _init():
        hf_s[...] = jnp.zeros_like(hf_s)
        cf_s[...] = jnp.zeros_like(cf_s)
        hb_s[...] = jnp.zeros_like(hb_s)
        cb_s[...] = jnp.zeros_like(cb_s)

    def body(t, _):
        hf, cf = _cellp(xf_ref[:, t, :] + _rec(hf_s[...], wf_ref),
                        cf_s[...], H)
        yf_ref[:, pl.ds(t, 1), :] = hf[:, None, :]
        hf_s[...] = hf
        cf_s[...] = cf
        tb = TC - 1 - t
        hb, cb = _cellp(xb_ref[:, tb, :] + _rec(hb_s[...], wb_ref),
                        cb_s[...], H)
        yb_ref[:, pl.ds(tb, 1), :] = hb[:, None, :]
        hb_s[...] = hb
        cb_s[...] = cb
        return 0

    jax.lax.fori_loop(0, TC, body, 0)


def _bilstm_big(x, p, chunks):
    B, T, _ = x.shape
    xf, xb, wf, wb = _lstm_proj(x, p)
    H = wf.shape[0]
    NC = chunks
    TC = T // NC
    yf, yb = pl.pallas_call(
        functools.partial(_bilstm_big_kern, TC=TC, H=H),
        grid=(NC,),
        in_specs=[
            pl.BlockSpec((B, TC, 4 * H), lambda i: (0, i, 0)),
            pl.BlockSpec((B, TC, 4 * H), lambda i, NC=NC: (0, NC - 1 - i, 0)),
            pl.BlockSpec((H, 4 * H), lambda i: (0, 0)),
            pl.BlockSpec((H, 4 * H), lambda i: (0, 0)),
        ],
        out_specs=[
            pl.BlockSpec((B, TC, H), lambda i: (0, i, 0)),
            pl.BlockSpec((B, TC, H), lambda i, NC=NC: (0, NC - 1 - i, 0)),
        ],
        out_shape=[jax.ShapeDtypeStruct((B, T, H), jnp.float32)] * 2,
        scratch_shapes=[pltpu.VMEM((B, H), jnp.float32)] * 4,
        compiler_params=pltpu.CompilerParams(
            dimension_semantics=("arbitrary",)),
    )(xf, xb, wf, wb)
    return jnp.concatenate([yf, yb], axis=-1)


def kernel(x_f0, x_org, c_trg, params):
    B, T, _ = x_f0.shape
    c = x_f0[:, :, :8]
    f = x_f0[:, :, 8:]
    for i in range(2):
        c = _conv1d(c, params['conv_c'][i]['W'], params['conv_c'][i]['b'])
        f = _conv1d(f, params['conv_f'][i]['W'], params['conv_f'][i]['b'])
    c = _conv1d(c, params['conv_c'][2]['W'], params['conv_c'][2]['b'],
                gn=params['gn_c'], gn_groups=32)
    f = _conv1d(f, params['conv_f'][2]['W'], params['conv_f'][2]['b'],
                gn=params['gn_f'], gn_groups=16)
    r = _conv1d(x_org, params['conv_r']['W'], params['conv_r']['b'],
                gn=params['gn_r'], gn_groups=8)
    c_out, f_out, r_out = _bilstm_multi(
        [c, f, r], [params['lstm_c'], params['lstm_f'], params['lstm_r']])
    codes_c = jnp.concatenate([c_out[:, 7::8, :8], c_out[:, ::8, 8:]], axis=-1)
    codes_f = jnp.concatenate([f_out[:, 7::8, :32], f_out[:, ::8, 32:]], axis=-1)
    codes_r = jnp.concatenate([r_out[:, 7::8, :1], r_out[:, ::8, 1:]], axis=-1)
    enc = jnp.concatenate([
        jnp.repeat(codes_c, 8, axis=1),
        jnp.repeat(codes_r, 8, axis=1),
        jnp.repeat(codes_f, 8, axis=1),
        jnp.broadcast_to(c_trg[:, None, :], (B, T, c_trg.shape[-1])),
    ], axis=-1)
    h = enc
    for layer in params['lstm_d']:
        h = _bilstm_big(h, layer, chunks=12)
    lin = params['linear']
    out = _matmul(h.reshape(B * T, h.shape[-1]), lin['W'].T, lin['b'])
    return out.reshape(B, T, lin['W'].shape[0])


# speedup vs baseline: 5.4958x; 1.1769x over previous
import functools

import jax
import jax.numpy as jnp
import numpy as np
from jax.experimental import pallas as pl
from jax.experimental.pallas import tpu as pltpu


def _mmdot(a, b):
    return jax.lax.dot_general(a, b, (((1,), (0,)), ((), ())),
                               preferred_element_type=jnp.float32)


def _mmdot_t(a, b):
    return jax.lax.dot_general(a, b, (((1,), (1,)), ((), ())),
                               preferred_element_type=jnp.float32)


def _gn_relu(acc, T, G, g_row, b_row, mcg, mgc):
    C = acc.shape[1]
    n = float((C // G) * T)
    s1 = jnp.sum(acc, axis=0, keepdims=True)
    s2 = jnp.sum(acc * acc, axis=0, keepdims=True)
    g1 = _mmdot(s1, mcg) / n
    g2 = _mmdot(s2, mcg) / n
    rstd = jax.lax.rsqrt(g2 - g1 * g1 + 1e-5)
    a = _mmdot(rstd, mgc) * g_row
    return jnp.maximum(acc * a + (b_row - _mmdot(g1, mgc) * a), 0.0)


def _conv5(xpad, wt, b_row):
    T = xpad.shape[0] - 4
    acc = None
    for k in range(5):
        z = _mmdot(xpad[k:k + T, :], wt[k])
        acc = z if acc is None else acc + z
    return jnp.maximum(acc + b_row, 0.0)


def _pad2(y):
    z = jnp.zeros((2, y.shape[1]), jnp.float32)
    return jnp.concatenate([z, y, z], axis=0)


def _enc_convs_kern(xcf_ref, xr_ref, wc1, wc2, wc3, bc, gnc,
                    wf1, wf2, wf3, bf, gnf, wr1, br, gnr,
                    mcg_c, mgc_c, mcg_f, mgc_f, mcg_r, mgc_r,
                    oc_ref, of_ref, or_ref, *, T):
    xc = xcf_ref[0, :, :8]
    xf = xcf_ref[0, :, 8:]
    c = _conv5(xc, wc1[...], bc[0][None, :])
    c = _conv5(_pad2(c), wc2[...], bc[1][None, :])
    c = _conv5(_pad2(c), wc3[...], bc[2][None, :])
    oc_ref[0] = _gn_relu(c, T, mcg_c.shape[1], gnc[0][None, :],
                         gnc[1][None, :], mcg_c[...], mgc_c[...])
    f = _conv5(xf, wf1[...], bf[0][None, :])
    f = _conv5(_pad2(f), wf2[...], bf[1][None, :])
    f = _conv5(_pad2(f), wf3[...], bf[2][None, :])
    of_ref[0] = _gn_relu(f, T, mcg_f.shape[1], gnf[0][None, :],
                         gnf[1][None, :], mcg_f[...], mgc_f[...])
    r = _conv5(xr_ref[0], wr1[...], br[0][None, :])
    or_ref[0] = _gn_relu(r, T, mcg_r.shape[1], gnr[0][None, :],
                         gnr[1][None, :], mcg_r[...], mgc_r[...])


def _enc_convs(x_f0, x_org, params):
    B, T, _ = x_f0.shape
    xcf = jnp.pad(x_f0, ((0, 0), (2, 2), (0, 0)))
    xr = jnp.pad(x_org, ((0, 0), (2, 2), (0, 0)))

    def wt(p):
        return jnp.transpose(p['W'], (2, 1, 0))

    def mk_m(C, G):
        m = jnp.repeat(jnp.eye(G, dtype=jnp.float32), C // G, axis=0)
        return m, m.T

    mcg_c, mgc_c = mk_m(512, 32)
    mcg_f, mgc_f = mk_m(256, 16)
    mcg_r, mgc_r = mk_m(128, 8)
    bc = jnp.stack([jnp.pad(params['conv_c'][i]['b'], (0, 0))
                    for i in range(3)])
    bf = jnp.stack([params['conv_f'][i]['b'] for i in range(3)])
    gnc = jnp.stack([params['gn_c']['g'], params['gn_c']['b']])
    gnf = jnp.stack([params['gn_f']['g'], params['gn_f']['b']])
    gnr = jnp.stack([params['gn_r']['g'], params['gn_r']['b']])
    br = params['conv_r']['b'].reshape(1, 128)
    ins = [xcf, xr,
           wt(params['conv_c'][0]), wt(params['conv_c'][1]),
           wt(params['conv_c'][2]), bc, gnc,
           wt(params['conv_f'][0]), wt(params['conv_f'][1]),
           wt(params['conv_f'][2]), bf, gnf,
           wt(params['conv_r']), br, gnr,
           mcg_c, mgc_c, mcg_f, mgc_f, mcg_r, mgc_r]
    in_specs = [
        pl.BlockSpec((1, T + 4, 265), lambda i: (i, 0, 0)),
        pl.BlockSpec((1, T + 4, 8), lambda i: (i, 0, 0)),
    ] + [pl.BlockSpec(tuple(int(d) for d in a.shape),
                      functools.partial(lambda nd, i: (0,) * nd, a.ndim))
         for a in ins[2:]]
    return pl.pallas_call(
        functools.partial(_enc_convs_kern, T=T),
        grid=(B,),
        in_specs=in_specs,
        out_specs=[pl.BlockSpec((1, T, C), lambda i: (i, 0, 0))
                   for C in (512, 256, 128)],
        out_shape=[jax.ShapeDtypeStruct((B, T, C), jnp.float32)
                   for C in (512, 256, 128)],
    )(*ins)


def _mm_kern(*refs, n, wt):
    o_ref, b_ref = refs[-1], refs[-2]
    z = b_ref[0][None, :]
    for k in range(n):
        if wt:
            z = z + _mmdot_t(refs[k][...], refs[n + k][...])
        else:
            z = z + _mmdot(refs[k][...], refs[n + k][...])
    o_ref[...] = z


def _matmul_multi(As, Ws, bias, wt=False):
    M = As[0].shape[0]
    N = Ws[0].shape[0] if wt else Ws[0].shape[1]
    n = len(As)
    bm = M if M <= 768 else 768
    assert M % bm == 0
    bn = N if N <= 2048 else 2048
    assert N % bn == 0
    in_specs = [pl.BlockSpec((bm, int(a.shape[1])),
                             lambda j, i: (i, 0)) for a in As]
    if wt:
        in_specs += [pl.BlockSpec((bn, int(w.shape[1])),
                                  lambda j, i: (j, 0)) for w in Ws]
    else:
        in_specs += [pl.BlockSpec((int(w.shape[0]), bn),
                                  lambda j, i: (0, j)) for w in Ws]
    in_specs += [pl.BlockSpec((1, bn), lambda j, i: (0, j))]
    return pl.pallas_call(
        functools.partial(_mm_kern, n=n, wt=wt),
        grid=(N // bn, M // bm),
        in_specs=in_specs,
        out_specs=pl.BlockSpec((bm, bn), lambda j, i: (i, j)),
        out_shape=jax.ShapeDtypeStruct((M, N), jnp.float32),
    )(*As, *Ws, bias.reshape(1, N))


def _matmul(a, w, bias):
    return _matmul_multi([a], [w], bias)


def _cellp(z, c, H):
    s = jax.nn.sigmoid(z[:, :2 * H])
    g = jnp.tanh(z[:, 2 * H:3 * H])
    o = jax.nn.sigmoid(z[:, 3 * H:])
    c = s[:, H:] * c + s[:, :H] * g
    h = o * jnp.tanh(c)
    return h, c


def _rec(h, w_ref):
    return _mmdot_t(h, w_ref[...])


def _bilstm_pack_kern(xp_ref, wf_ref, wb_ref, cof_ref, cob_ref, *, T, HT):
    B = xp_ref.shape[0]
    zero = jnp.zeros((B, HT), jnp.float32)

    def step(t, carry):
        hf, cf, hb, cb = carry
        tb = T - 1 - t
        zf = xp_ref[:, t, :4 * HT] + _rec(hf, wf_ref)
        hf, cf = _cellp(zf, cf, HT)
        zb = xp_ref[:, tb, 4 * HT:] + _rec(hb, wb_ref)
        hb, cb = _cellp(zb, cb, HT)
        return hf, cf, hb, cb

    def body(k, carry):
        for u in range(8):
            carry = step(8 * k + u, carry)
        hf, cf, hb, cb = carry
        cof_ref[pl.ds(k, 1), :, :] = hf[None, :, :]
        cob_ref[pl.ds(T // 8 - 1 - k, 1), :, :] = hb[None, :, :]
        return carry

    jax.lax.fori_loop(0, T // 8, body, (zero, zero, zero, zero))


def _bilstm_multi(xs, ps):
    B, T = xs[0].shape[:2]
    Hs = [int(p['fwd'][1].shape[1]) for p in ps]
    Is = [int(x.shape[-1]) for x in xs]
    HT, IT = 128, sum(Is)
    offH = np.concatenate([[0], np.cumsum(Hs)])
    offI = np.concatenate([[0], np.cumsum(Is)])
    WallTs = [jnp.zeros((8 * HT, I), jnp.float32) for I in Is]
    ball = jnp.zeros((8 * HT,), jnp.float32)
    Whh_d = [jnp.zeros((4 * HT, HT), jnp.float32) for _ in range(2)]
    for k, p in enumerate(ps):
        H = Hs[k]
        for d, dirn in enumerate(('fwd', 'bwd')):
            Wih, Whh, bih, bhh = p[dirn]
            bb = bih + bhh
            for q in range(4):
                row = d * 4 * HT + q * HT + offH[k]
                WallTs[k] = WallTs[k].at[row:row + H, :].set(
                    Wih[q * H:(q + 1) * H])
                ball = ball.at[row:row + H].set(bb[q * H:(q + 1) * H])
                rrow = q * HT + offH[k]
                Whh_d[d] = Whh_d[d].at[rrow:rrow + H,
                                       offH[k]:offH[k + 1]].set(
                    Whh[q * H:(q + 1) * H])
    xp = _matmul_multi([x.reshape(B * T, -1) for x in xs], WallTs, ball,
                       wt=True).reshape(B, T, 8 * HT)
    ins = (xp, Whh_d[0], Whh_d[1])
    cof, cob = pl.pallas_call(
        functools.partial(_bilstm_pack_kern, T=T, HT=HT),
        in_specs=[pl.BlockSpec(a.shape, functools.partial(
                      lambda nd: (0,) * nd, a.ndim)) for a in ins],
        out_specs=[pl.BlockSpec((T // 8, B, HT), lambda: (0, 0, 0))] * 2,
        out_shape=[jax.ShapeDtypeStruct((T // 8, B, HT), jnp.float32)] * 2,
    )(*ins)
    return cof, cob, offH


def _dec_kern(*refs, TC, H, NP, TMAJ, IOFF):
    xfs = refs[0:NP]
    xbs = refs[NP:2 * NP]
    (wihf_ref, wihb_ref, bf_ref, bb_ref, whf_ref, whb_ref, yf_ref, yb_ref,
     hf_s, cf_s, hb_s, cb_s, xpf_s, xpb_s) = refs[2 * NP:]
    B = hf_s.shape[0]

    @pl.when(pl.program_id(0) == 0)
    def _init():
        hf_s[...] = jnp.zeros_like(hf_s)
        cf_s[...] = jnp.zeros_like(cf_s)
        hb_s[...] = jnp.zeros_like(hb_s)
        cb_s[...] = jnp.zeros_like(cb_s)

    def proj(parts, wref, b_ref):
        acc = None
        for k in range(NP):
            w = wref[:, IOFF[k]:IOFF[k + 1]]
            if TMAJ:
                TCp, _, Ik = parts[k].shape
                z = _mmdot_t(parts[k][...].reshape(TCp * B, Ik), w)
                z = z.reshape(TCp, B, 4 * H)
                r = TC // TCp
                if r > 1:
                    z = jnp.broadcast_to(z[:, None, :, :],
                                         (TCp, r, B, 4 * H))
                    z = z.reshape(TC, B, 4 * H)
            else:
                TCp, Ik = parts[k].shape[1], parts[k].shape[2]
                z = _mmdot_t(parts[k][...].reshape(B * TCp, Ik), w)
                z = z.reshape(B, TC, 4 * H)
            acc = z if acc is None else acc + z
        return acc + b_ref[0][None, None, :]

    xpf_s[...] = proj(xfs, wihf_ref, bf_ref)
    xpb_s[...] = proj(xbs, wihb_ref, bb_ref)

    def xp_at(s, t):
        return s[t] if TMAJ else s[:, t, :]

    def step(t):
        hf, cf = _cellp(xp_at(xpf_s, t) + _rec(hf_s[...], whf_ref),
                        cf_s[...], H)
        yf_ref[:, pl.ds(t, 1), :] = hf[:, None, :]
        hf_s[...] = hf
        cf_s[...] = cf
        tb = TC - 1 - t
        hb, cb = _cellp(xp_at(xpb_s, tb) + _rec(hb_s[...], whb_ref),
                        cb_s[...], H)
        yb_ref[:, pl.ds(tb, 1), :] = hb[:, None, :]
        hb_s[...] = hb
        cb_s[...] = cb

    def body(t, _):
        for u in range(8):
            step(8 * t + u)
        return 0

    jax.lax.fori_loop(0, TC // 8, body, 0)


def _bilstm_big(xs, wih, p, chunks, T, tmaj=False):
    B = xs[0].shape[1] if tmaj else xs[0].shape[0]
    H = int(p['fwd'][1].shape[1])
    NP = len(xs)
    NC = chunks
    TC = T // NC
    IOFF = tuple(np.concatenate(
        [[0], np.cumsum([int(x.shape[-1]) for x in xs])]).tolist())
    bfv = (p['fwd'][2] + p['fwd'][3]).reshape(1, 4 * H)
    bbv = (p['bwd'][2] + p['bwd'][3]).reshape(1, 4 * H)

    def chunk_spec(x, back):
        w = int(x.shape[-1])
        if tmaj:
            Tp = x.shape[0]
            if Tp == 1:
                return pl.BlockSpec((1, B, w), lambda i: (0, 0, 0))
            TCp = Tp // NC
            if back:
                return pl.BlockSpec((TCp, B, w),
                                    lambda i, NC=NC: (NC - 1 - i, 0, 0))
            return pl.BlockSpec((TCp, B, w), lambda i: (i, 0, 0))
        if back:
            return pl.BlockSpec((B, TC, w),
                                lambda i, NC=NC: (0, NC - 1 - i, 0))
        return pl.BlockSpec((B, TC, w), lambda i: (0, i, 0))

    ins = (list(xs) + list(xs)
           + [wih[0], wih[1], bfv, bbv, p['fwd'][1], p['bwd'][1]])
    in_specs = (
        [chunk_spec(x, False) for x in xs]
        + [chunk_spec(x, True) for x in xs]
        + [pl.BlockSpec(tuple(int(d) for d in w.shape), lambda i: (0, 0))
           for w in (wih[0], wih[1])]
        + [pl.BlockSpec((1, 4 * H), lambda i: (0, 0))] * 2
        + [pl.BlockSpec((4 * H, H), lambda i: (0, 0))] * 2
    )
    yf, yb = pl.pallas_call(
        functools.partial(_dec_kern, TC=TC, H=H, NP=NP, TMAJ=tmaj,
                          IOFF=IOFF),
        grid=(NC,),
        in_specs=in_specs,
        out_specs=[
            pl.BlockSpec((B, TC, H), lambda i: (0, i, 0)),
            pl.BlockSpec((B, TC, H), lambda i, NC=NC: (0, NC - 1 - i, 0)),
        ],
        out_shape=[jax.ShapeDtypeStruct((B, T, H), jnp.float32)] * 2,
        scratch_shapes=[pltpu.VMEM((B, H), jnp.float32)] * 4
        + [pltpu.VMEM((TC, B, 4 * H) if tmaj else
                      (B, TC, 4 * H), jnp.float32)] * 2,
        compiler_params=pltpu.CompilerParams(
            dimension_semantics=("arbitrary",)),
    )(*ins)
    return yf, yb


def kernel(x_f0, x_org, c_trg, params):
    B, T, _ = x_f0.shape
    c, f, r = _enc_convs(x_f0, x_org, params)
    cof, cob, offH = _bilstm_multi(
        [c, f, r], [params['lstm_c'], params['lstm_f'], params['lstm_r']])
    def l1_w(Wih):
        w = jnp.zeros((2048, 384), jnp.float32)
        w = w.at[:, 0:8].set(Wih[:, 0:8])
        w = w.at[:, 8:40].set(Wih[:, 18:50])
        w = w.at[:, 40:41].set(Wih[:, 16:17])
        w = w.at[:, 128:136].set(Wih[:, 8:16])
        w = w.at[:, 136:168].set(Wih[:, 50:82])
        w = w.at[:, 168:169].set(Wih[:, 17:18])
        w = w.at[:, 256:338].set(Wih[:, 82:164])
        return w

    ct_tm = jnp.pad(c_trg, ((0, 0), (0, 46)))[None]
    l1 = params['lstm_d'][0]
    hs = list(_bilstm_big(
        [cof, cob, ct_tm], (l1_w(l1['fwd'][0]), l1_w(l1['bwd'][0])),
        l1, chunks=12, T=T, tmaj=True))
    for layer in params['lstm_d'][1:]:
        hs = list(_bilstm_big(hs, (layer['fwd'][0], layer['bwd'][0]),
                              layer, chunks=12, T=T))
    lin = params['linear']
    out = _matmul_multi(
        [h.reshape(B * T, h.shape[-1]) for h in hs],
        [lin['W'][:, :512], lin['W'][:, 512:]], lin['b'], wt=True)
    return out.reshape(B, T, lin['W'].shape[0])


# speedup vs baseline: 5.5090x; 1.0024x over previous
import functools

import jax
import jax.numpy as jnp
import numpy as np
from jax.experimental import pallas as pl
from jax.experimental.pallas import tpu as pltpu


def _mmdot(a, b):
    return jax.lax.dot_general(a, b, (((1,), (0,)), ((), ())),
                               preferred_element_type=jnp.float32)


def _conv_kern(x_ref, w_ref, b_ref, o_ref, *, K, T, gn_groups, g_ref=None,
               bt_ref=None, mcg_ref=None, mgc_ref=None):
    acc = jnp.zeros((T, w_ref.shape[2]), jnp.float32)
    for k in range(K):
        acc += _mmdot(x_ref[0, k:k + T, :], w_ref[k])
    acc = jnp.maximum(acc + b_ref[0][None, :], 0.0)
    if gn_groups:
        C = acc.shape[1]
        cs = C // gn_groups
        n = float(cs * T)
        s1 = jnp.sum(acc, axis=0, keepdims=True)
        s2 = jnp.sum(acc * acc, axis=0, keepdims=True)
        g1 = _mmdot(s1, mcg_ref[...]) / n
        g2 = _mmdot(s2, mcg_ref[...]) / n
        rstd = jax.lax.rsqrt(g2 - g1 * g1 + 1e-5)
        mu_c = _mmdot(g1, mgc_ref[...])
        rs_c = _mmdot(rstd, mgc_ref[...])
        a = rs_c * g_ref[0][None, :]
        acc = jnp.maximum(acc * a + (bt_ref[0][None, :] - mu_c * a), 0.0)
    o_ref[0] = acc


def _conv1d(x, W, b, gn=None, gn_groups=0):
    B, T, cin = x.shape
    cout, _, K = W.shape
    pad = (K - 1) // 2
    xp = jnp.pad(x, ((0, 0), (pad, pad), (0, 0)))
    wt = jnp.transpose(W, (2, 1, 0))
    b2 = b.reshape(1, cout)
    ins = [xp, wt, b2]
    in_specs = [
        pl.BlockSpec((1, T + K - 1, cin), lambda i: (i, 0, 0)),
        pl.BlockSpec((K, cin, cout), lambda i: (0, 0, 0)),
        pl.BlockSpec((1, cout), lambda i: (0, 0)),
    ]
    if gn_groups:
        mcg = jnp.repeat(jnp.eye(gn_groups, dtype=jnp.float32),
                         cout // gn_groups, axis=0)
        ins += [gn['g'].reshape(1, cout), gn['b'].reshape(1, cout),
                mcg, mcg.T]
        in_specs += [pl.BlockSpec((1, cout), lambda i: (0, 0))] * 2 + [
            pl.BlockSpec((cout, gn_groups), lambda i: (0, 0)),
            pl.BlockSpec((gn_groups, cout), lambda i: (0, 0)),
        ]
        kern = functools.partial(_conv_kern, K=K, T=T, gn_groups=gn_groups)

        def wrapped(x_r, w_r, b_r, g_r, bt_r, mcg_r, mgc_r, o_r):
            kern(x_r, w_r, b_r, o_r, g_ref=g_r, bt_ref=bt_r,
                 mcg_ref=mcg_r, mgc_ref=mgc_r)
        body = wrapped
    else:
        body = functools.partial(_conv_kern, K=K, T=T, gn_groups=0)
    return pl.pallas_call(
        body,
        grid=(B,),
        in_specs=in_specs,
        out_specs=pl.BlockSpec((1, T, cout), lambda i: (i, 0, 0)),
        out_shape=jax.ShapeDtypeStruct((B, T, cout), jnp.float32),
    )(*ins)


def _mm_kern(*refs, n):
    o_ref, b_ref = refs[-1], refs[-2]
    z = b_ref[0][None, :]
    for k in range(n):
        z = z + _mmdot(refs[k][...], refs[n + k][...])
    o_ref[...] = z


def _matmul_multi(As, Ws, bias):
    M = As[0].shape[0]
    N = Ws[0].shape[1]
    n = len(As)
    bm = M if M <= 768 else 768
    assert M % bm == 0
    bn = N if N <= 2048 else 2048
    assert N % bn == 0
    in_specs = [pl.BlockSpec((bm, int(a.shape[1])),
                             lambda j, i: (i, 0)) for a in As]
    in_specs += [pl.BlockSpec((int(w.shape[0]), bn),
                              lambda j, i: (0, j)) for w in Ws]
    in_specs += [pl.BlockSpec((1, bn), lambda j, i: (0, j))]
    return pl.pallas_call(
        functools.partial(_mm_kern, n=n),
        grid=(N // bn, M // bm),
        in_specs=in_specs,
        out_specs=pl.BlockSpec((bm, bn), lambda j, i: (i, j)),
        out_shape=jax.ShapeDtypeStruct((M, N), jnp.float32),
    )(*As, *Ws, bias.reshape(1, N))


def _matmul(a, w, bias):
    return _matmul_multi([a], [w], bias)


_GSRC = (0, 1, 3, 2)


def _permg(w4, H):
    return jnp.concatenate([w4[..., q * H:(q + 1) * H] for q in _GSRC],
                           axis=-1)


def _cellp(z, c, H):
    s = jax.nn.sigmoid(z[:, :3 * H])
    g = jnp.tanh(z[:, 3 * H:])
    c = s[:, H:2 * H] * c + s[:, :H] * g
    h = s[:, 2 * H:3 * H] * jnp.tanh(c)
    return h, c


def _rec(h, w_ref):
    return jax.lax.dot_general(h, w_ref[...], (((1,), (0,)), ((), ())),
                               preferred_element_type=jnp.float32)


def _bilstm_pack_kern(xf_ref, xb_ref, wf_ref, wb_ref, cof_ref, cob_ref, *,
                      T, HT):
    B = xf_ref.shape[0]
    zero = jnp.zeros((B, HT), jnp.float32)

    def step(t, carry):
        hf, cf, hb, cb = carry
        tb = T - 1 - t
        hf, cf = _cellp(xf_ref[:, t, :] + _rec(hf, wf_ref), cf, HT)
        hb, cb = _cellp(xb_ref[:, tb, :] + _rec(hb, wb_ref), cb, HT)
        return hf, cf, hb, cb

    def body(k, carry):
        for u in range(8):
            carry = step(8 * k + u, carry)
        hf, cf, hb, cb = carry
        cof_ref[pl.ds(k, 1), :, :] = hf[None, :, :]
        cob_ref[pl.ds(T // 8 - 1 - k, 1), :, :] = hb[None, :, :]
        return carry

    jax.lax.fori_loop(0, T // 8, body, (zero, zero, zero, zero))


def _bilstm_multi(xs, ps):
    B, T = xs[0].shape[:2]
    Hs = [int(p['fwd'][1].shape[1]) for p in ps]
    Is = [int(x.shape[-1]) for x in xs]
    HT, IT = 128, sum(Is)
    offH = np.concatenate([[0], np.cumsum(Hs)])
    offI = np.concatenate([[0], np.cumsum(Is)])
    Wall = jnp.zeros((IT, 8 * HT), jnp.float32)
    ball = jnp.zeros((8 * HT,), jnp.float32)
    Whh_d = [jnp.zeros((HT, 4 * HT), jnp.float32) for _ in range(2)]
    for k, p in enumerate(ps):
        H = Hs[k]
        for d, dirn in enumerate(('fwd', 'bwd')):
            Wih, Whh, bih, bhh = p[dirn]
            WT, HhT, bb = Wih.T, Whh.T, bih + bhh
            for q in range(4):
                sq = _GSRC[q]
                col = d * 4 * HT + q * HT + offH[k]
                Wall = Wall.at[offI[k]:offI[k + 1], col:col + H].set(
                    WT[:, sq * H:(sq + 1) * H])
                ball = ball.at[col:col + H].set(bb[sq * H:(sq + 1) * H])
                rcol = q * HT + offH[k]
                Whh_d[d] = Whh_d[d].at[offH[k]:offH[k + 1],
                                       rcol:rcol + H].set(
                    HhT[:, sq * H:(sq + 1) * H])
    xcat = jnp.concatenate(xs, axis=-1).reshape(B * T, IT)
    xp = _matmul(xcat, Wall, ball).reshape(B, T, 8 * HT)
    xf, xb = xp[..., :4 * HT], xp[..., 4 * HT:]
    ins = (xf, xb, Whh_d[0], Whh_d[1])
    cof, cob = pl.pallas_call(
        functools.partial(_bilstm_pack_kern, T=T, HT=HT),
        in_specs=[pl.BlockSpec(a.shape, functools.partial(
                      lambda nd: (0,) * nd, a.ndim)) for a in ins],
        out_specs=[pl.BlockSpec((T // 8, B, HT), lambda: (0, 0, 0))] * 2,
        out_shape=[jax.ShapeDtypeStruct((T // 8, B, HT), jnp.float32)] * 2,
    )(*ins)
    return cof, cob, offH


def _dec_kern(*refs, TC, H, NP, HASC):
    xfs = refs[0:NP]
    xbs = refs[NP:2 * NP]
    wihf = refs[2 * NP:3 * NP]
    wihb = refs[3 * NP:4 * NP]
    i0 = 4 * NP
    if HASC:
        ctf_ref, ctb_ref = refs[i0:i0 + 2]
        i0 += 2
    (bf_ref, bb_ref, whf_ref, whb_ref, yf_ref, yb_ref,
     hf_s, cf_s, hb_s, cb_s, xpf_s, xpb_s) = refs[i0:]
    B = hf_s.shape[0]

    @pl.when(pl.program_id(0) == 0)
    def _init():
        hf_s[...] = jnp.zeros_like(hf_s)
        cf_s[...] = jnp.zeros_like(cf_s)
        hb_s[...] = jnp.zeros_like(hb_s)
        cb_s[...] = jnp.zeros_like(cb_s)

    def proj(parts, ws, b_ref, ct_ref):
        acc = None
        for k in range(NP):
            if HASC:
                TCp, _, Ik = parts[k].shape
                z = _mmdot(parts[k][...].reshape(TCp * B, Ik), ws[k][...])
                z = z.reshape(TCp, B, 4 * H)
                r = TC // TCp
                if r > 1:
                    z = jnp.broadcast_to(z[:, None, :, :],
                                         (TCp, r, B, 4 * H))
                    z = z.reshape(TC, B, 4 * H)
            else:
                TCp, Ik = parts[k].shape[1], parts[k].shape[2]
                z = _mmdot(parts[k][...].reshape(B * TCp, Ik), ws[k][...])
                z = z.reshape(B, TC, 4 * H)
            acc = z if acc is None else acc + z
        acc = acc + b_ref[0][None, None, :]
        if ct_ref is not None:
            acc = acc + ct_ref[...][None, :, :]
        return acc

    xpf_s[...] = proj(xfs, wihf, bf_ref, ctf_ref if HASC else None)
    xpb_s[...] = proj(xbs, wihb, bb_ref, ctb_ref if HASC else None)

    def xp_at(s, t):
        return s[t] if HASC else s[:, t, :]

    def step(t):
        hf, cf = _cellp(xp_at(xpf_s, t) + _rec(hf_s[...], whf_ref),
                        cf_s[...], H)
        yf_ref[:, pl.ds(t, 1), :] = hf[:, None, :]
        hf_s[...] = hf
        cf_s[...] = cf
        tb = TC - 1 - t
        hb, cb = _cellp(xp_at(xpb_s, tb) + _rec(hb_s[...], whb_ref),
                        cb_s[...], H)
        yb_ref[:, pl.ds(tb, 1), :] = hb[:, None, :]
        hb_s[...] = hb
        cb_s[...] = cb

    def body(t, _):
        for u in range(8):
            step(8 * t + u)
        return 0

    jax.lax.fori_loop(0, TC // 8, body, 0)


def _bilstm_big(xs, Wparts, p, chunks, T, cts=None):
    B = xs[0].shape[1] if cts is not None else xs[0].shape[0]
    H = int(p['fwd'][1].shape[1])
    NP = len(xs)
    NC = chunks
    TC = T // NC
    bfv = _permg(p['fwd'][3] + p['fwd'][2], H).reshape(1, 4 * H)
    bbv = _permg(p['bwd'][3] + p['bwd'][2], H).reshape(1, 4 * H)
    whf = _permg(p['fwd'][1].T, H)
    whb = _permg(p['bwd'][1].T, H)

    def chunk_spec(x, back):
        if cts is not None:
            TCp = TC * x.shape[0] // T
            if back:
                return pl.BlockSpec((TCp, B, int(x.shape[-1])),
                                    lambda i, NC=NC: (NC - 1 - i, 0, 0))
            return pl.BlockSpec((TCp, B, int(x.shape[-1])),
                                lambda i: (i, 0, 0))
        if back:
            return pl.BlockSpec((B, TC, int(x.shape[-1])),
                                lambda i, NC=NC: (0, NC - 1 - i, 0))
        return pl.BlockSpec((B, TC, int(x.shape[-1])),
                            lambda i: (0, i, 0))

    ins = (list(xs) + list(xs) + list(Wparts[0]) + list(Wparts[1])
           + (list(cts) if cts else [])
           + [bfv, bbv, whf, whb])
    in_specs = (
        [chunk_spec(x, False) for x in xs]
        + [chunk_spec(x, True) for x in xs]
        + [pl.BlockSpec((int(w.shape[0]), 4 * H), lambda i: (0, 0))
           for w in list(Wparts[0]) + list(Wparts[1])]
        + ([pl.BlockSpec((B, 4 * H), lambda i: (0, 0))] * 2 if cts else [])
        + [pl.BlockSpec((1, 4 * H), lambda i: (0, 0))] * 2
        + [pl.BlockSpec((H, 4 * H), lambda i: (0, 0))] * 2
    )
    yf, yb = pl.pallas_call(
        functools.partial(_dec_kern, TC=TC, H=H, NP=NP,
                          HASC=cts is not None),
        grid=(NC,),
        in_specs=in_specs,
        out_specs=[
            pl.BlockSpec((B, TC, H), lambda i: (0, i, 0)),
            pl.BlockSpec((B, TC, H), lambda i, NC=NC: (0, NC - 1 - i, 0)),
        ],
        out_shape=[jax.ShapeDtypeStruct((B, T, H), jnp.float32)] * 2,
        scratch_shapes=[pltpu.VMEM((B, H), jnp.float32)] * 4
        + [pltpu.VMEM((TC, B, 4 * H) if cts is not None else
                      (B, TC, 4 * H), jnp.float32)] * 2,
        compiler_params=pltpu.CompilerParams(
            dimension_semantics=("arbitrary",)),
    )(*ins)
    return yf, yb


def kernel(x_f0, x_org, c_trg, params):
    B, T, _ = x_f0.shape
    c = x_f0[:, :, :8]
    f = x_f0[:, :, 8:]
    for i in range(2):
        c = _conv1d(c, params['conv_c'][i]['W'], params['conv_c'][i]['b'])
        f = _conv1d(f, params['conv_f'][i]['W'], params['conv_f'][i]['b'])
    c = _conv1d(c, params['conv_c'][2]['W'], params['conv_c'][2]['b'],
                gn=params['gn_c'], gn_groups=32)
    f = _conv1d(f, params['conv_f'][2]['W'], params['conv_f'][2]['b'],
                gn=params['gn_f'], gn_groups=16)
    r = _conv1d(x_org, params['conv_r']['W'], params['conv_r']['b'],
                gn=params['gn_r'], gn_groups=8)
    cof, cob, offH = _bilstm_multi(
        [c, f, r], [params['lstm_c'], params['lstm_f'], params['lstm_r']])
    def l1_w(dirp):
        Wih = dirp[0]
        H = 512
        WT = _permg(Wih.T, H)
        wf_rows = jnp.zeros((128, 4 * H), jnp.float32)
        wf_rows = wf_rows.at[0:8].set(WT[0:8])
        wf_rows = wf_rows.at[8:40].set(WT[18:50])
        wf_rows = wf_rows.at[40:41].set(WT[16:17])
        wb_rows = jnp.zeros((128, 4 * H), jnp.float32)
        wb_rows = wb_rows.at[0:8].set(WT[8:16])
        wb_rows = wb_rows.at[8:40].set(WT[50:82])
        wb_rows = wb_rows.at[40:41].set(WT[17:18])
        wct = WT[82:164]
        ct = _matmul(c_trg, wct, jnp.zeros((4 * H,), jnp.float32))
        return wf_rows, wb_rows, ct

    l1f_c, l1f_b, ctf = l1_w(params['lstm_d'][0]['fwd'])
    l1b_c, l1b_b, ctb = l1_w(params['lstm_d'][0]['bwd'])
    hs = list(_bilstm_big(
        [cof, cob], ([l1f_c, l1f_b], [l1b_c, l1b_b]),
        params['lstm_d'][0], chunks=12, T=T, cts=(ctf, ctb)))
    for layer in params['lstm_d'][1:]:
        offs = [0, 512, 1024]
        Wp = []
        for dirn in ('fwd', 'bwd'):
            WT = _permg(layer[dirn][0].T, 512)
            Wp.append([WT[offs[k]:offs[k + 1]] for k in range(2)])
        hs = list(_bilstm_big(hs, (Wp[0], Wp[1]), layer, chunks=12, T=T))
    lin = params['linear']
    WT = lin['W'].T
    out = _matmul_multi(
        [h.reshape(B * T, h.shape[-1]) for h in hs],
        [WT[:512], WT[512:]], lin['b'])
    return out.reshape(B, T, lin['W'].shape[0])


# speedup vs baseline: 5.5196x; 1.0019x over previous
import functools

import jax
import jax.numpy as jnp
import numpy as np
from jax.experimental import pallas as pl
from jax.experimental.pallas import tpu as pltpu


def _mmdot(a, b):
    return jax.lax.dot_general(a, b, (((1,), (0,)), ((), ())),
                               preferred_element_type=jnp.float32)


def _conv_kern(x_ref, w_ref, b_ref, o_ref, *, K, T, gn_groups, g_ref=None,
               bt_ref=None, mcg_ref=None, mgc_ref=None):
    acc = jnp.zeros((T, w_ref.shape[2]), jnp.float32)
    for k in range(K):
        acc += _mmdot(x_ref[0, k:k + T, :], w_ref[k])
    acc = jnp.maximum(acc + b_ref[0][None, :], 0.0)
    if gn_groups:
        C = acc.shape[1]
        cs = C // gn_groups
        n = float(cs * T)
        s1 = jnp.sum(acc, axis=0, keepdims=True)
        s2 = jnp.sum(acc * acc, axis=0, keepdims=True)
        g1 = _mmdot(s1, mcg_ref[...]) / n
        g2 = _mmdot(s2, mcg_ref[...]) / n
        rstd = jax.lax.rsqrt(g2 - g1 * g1 + 1e-5)
        mu_c = _mmdot(g1, mgc_ref[...])
        rs_c = _mmdot(rstd, mgc_ref[...])
        a = rs_c * g_ref[0][None, :]
        acc = jnp.maximum(acc * a + (bt_ref[0][None, :] - mu_c * a), 0.0)
    o_ref[0] = acc


def _conv1d(x, W, b, gn=None, gn_groups=0):
    B, T, cin = x.shape
    cout, _, K = W.shape
    pad = (K - 1) // 2
    xp = jnp.pad(x, ((0, 0), (pad, pad), (0, 0)))
    wt = jnp.transpose(W, (2, 1, 0))
    b2 = b.reshape(1, cout)
    ins = [xp, wt, b2]
    in_specs = [
        pl.BlockSpec((1, T + K - 1, cin), lambda i: (i, 0, 0)),
        pl.BlockSpec((K, cin, cout), lambda i: (0, 0, 0)),
        pl.BlockSpec((1, cout), lambda i: (0, 0)),
    ]
    if gn_groups:
        mcg = jnp.repeat(jnp.eye(gn_groups, dtype=jnp.float32),
                         cout // gn_groups, axis=0)
        ins += [gn['g'].reshape(1, cout), gn['b'].reshape(1, cout),
                mcg, mcg.T]
        in_specs += [pl.BlockSpec((1, cout), lambda i: (0, 0))] * 2 + [
            pl.BlockSpec((cout, gn_groups), lambda i: (0, 0)),
            pl.BlockSpec((gn_groups, cout), lambda i: (0, 0)),
        ]
        kern = functools.partial(_conv_kern, K=K, T=T, gn_groups=gn_groups)

        def wrapped(x_r, w_r, b_r, g_r, bt_r, mcg_r, mgc_r, o_r):
            kern(x_r, w_r, b_r, o_r, g_ref=g_r, bt_ref=bt_r,
                 mcg_ref=mcg_r, mgc_ref=mgc_r)
        body = wrapped
    else:
        body = functools.partial(_conv_kern, K=K, T=T, gn_groups=0)
    return pl.pallas_call(
        body,
        grid=(B,),
        in_specs=in_specs,
        out_specs=pl.BlockSpec((1, T, cout), lambda i: (i, 0, 0)),
        out_shape=jax.ShapeDtypeStruct((B, T, cout), jnp.float32),
    )(*ins)


def _mm_kern(*refs, n):
    o_ref, b_ref = refs[-1], refs[-2]
    z = b_ref[0][None, :]
    for k in range(n):
        z = z + _mmdot(refs[k][...], refs[n + k][...])
    o_ref[...] = z


def _matmul_multi(As, Ws, bias):
    M = As[0].shape[0]
    N = Ws[0].shape[1]
    n = len(As)
    bm = M if M <= 768 else 768
    assert M % bm == 0
    bn = N if N <= 2048 else 2048
    assert N % bn == 0
    in_specs = [pl.BlockSpec((bm, int(a.shape[1])),
                             lambda j, i: (i, 0)) for a in As]
    in_specs += [pl.BlockSpec((int(w.shape[0]), bn),
                              lambda j, i: (0, j)) for w in Ws]
    in_specs += [pl.BlockSpec((1, bn), lambda j, i: (0, j))]
    return pl.pallas_call(
        functools.partial(_mm_kern, n=n),
        grid=(N // bn, M // bm),
        in_specs=in_specs,
        out_specs=pl.BlockSpec((bm, bn), lambda j, i: (i, j)),
        out_shape=jax.ShapeDtypeStruct((M, N), jnp.float32),
    )(*As, *Ws, bias.reshape(1, N))


def _matmul(a, w, bias):
    return _matmul_multi([a], [w], bias)


_GSRC = (0, 1, 3, 2)


def _permg(w4, H):
    return jnp.concatenate([w4[..., q * H:(q + 1) * H] for q in _GSRC],
                           axis=-1)


def _cellp(z, c, H):
    s = jax.nn.sigmoid(z[:, :3 * H])
    g = jnp.tanh(z[:, 3 * H:])
    c = s[:, H:2 * H] * c + s[:, :H] * g
    h = s[:, 2 * H:3 * H] * jnp.tanh(c)
    return h, c


def _rec(h, w_ref):
    return jax.lax.dot_general(h, w_ref[...], (((1,), (0,)), ((), ())),
                               preferred_element_type=jnp.float32)


def _bilstm_pack_kern(xf_ref, xb_ref, wf_ref, wb_ref, cof_ref, cob_ref, *,
                      T, HT):
    B = xf_ref.shape[0]
    zero = jnp.zeros((B, HT), jnp.float32)

    def step(t, carry):
        hf, cf, hb, cb = carry
        tb = T - 1 - t
        hf, cf = _cellp(xf_ref[:, t, :] + _rec(hf, wf_ref), cf, HT)
        hb, cb = _cellp(xb_ref[:, tb, :] + _rec(hb, wb_ref), cb, HT)
        return hf, cf, hb, cb

    def body(k, carry):
        for u in range(8):
            carry = step(8 * k + u, carry)
        hf, cf, hb, cb = carry
        cof_ref[pl.ds(k, 1), :, :] = hf[None, :, :]
        cob_ref[pl.ds(T // 8 - 1 - k, 1), :, :] = hb[None, :, :]
        return carry

    jax.lax.fori_loop(0, T // 8, body, (zero, zero, zero, zero))


def _bilstm_multi(xs, ps):
    B, T = xs[0].shape[:2]
    Hs = [int(p['fwd'][1].shape[1]) for p in ps]
    Is = [int(x.shape[-1]) for x in xs]
    HT, IT = 128, sum(Is)
    offH = np.concatenate([[0], np.cumsum(Hs)])
    offI = np.concatenate([[0], np.cumsum(Is)])
    Wall = jnp.zeros((IT, 8 * HT), jnp.float32)
    ball = jnp.zeros((8 * HT,), jnp.float32)
    Whh_d = [jnp.zeros((HT, 4 * HT), jnp.float32) for _ in range(2)]
    for k, p in enumerate(ps):
        H = Hs[k]
        for d, dirn in enumerate(('fwd', 'bwd')):
            Wih, Whh, bih, bhh = p[dirn]
            WT, HhT, bb = Wih.T, Whh.T, bih + bhh
            for q in range(4):
                sq = _GSRC[q]
                col = d * 4 * HT + q * HT + offH[k]
                Wall = Wall.at[offI[k]:offI[k + 1], col:col + H].set(
                    WT[:, sq * H:(sq + 1) * H])
                ball = ball.at[col:col + H].set(bb[sq * H:(sq + 1) * H])
                rcol = q * HT + offH[k]
                Whh_d[d] = Whh_d[d].at[offH[k]:offH[k + 1],
                                       rcol:rcol + H].set(
                    HhT[:, sq * H:(sq + 1) * H])
    xcat = jnp.concatenate(xs, axis=-1).reshape(B * T, IT)
    xp = _matmul(xcat, Wall, ball).reshape(B, T, 8 * HT)
    xf, xb = xp[..., :4 * HT], xp[..., 4 * HT:]
    ins = (xf, xb, Whh_d[0], Whh_d[1])
    cof, cob = pl.pallas_call(
        functools.partial(_bilstm_pack_kern, T=T, HT=HT),
        in_specs=[pl.BlockSpec(a.shape, functools.partial(
                      lambda nd: (0,) * nd, a.ndim)) for a in ins],
        out_specs=[pl.BlockSpec((T // 8, B, HT), lambda: (0, 0, 0))] * 2,
        out_shape=[jax.ShapeDtypeStruct((T // 8, B, HT), jnp.float32)] * 2,
    )(*ins)
    return cof, cob, offH


def _dec_kern(*refs, TC, H, NP, HASC):
    xfs = refs[0:NP]
    xbs = refs[NP:2 * NP]
    wihf = refs[2 * NP:3 * NP]
    wihb = refs[3 * NP:4 * NP]
    i0 = 4 * NP
    if HASC:
        ctf_ref, ctb_ref = refs[i0:i0 + 2]
        i0 += 2
    (bf_ref, bb_ref, whf_ref, whb_ref, yf_ref, yb_ref,
     hf_s, cf_s, hb_s, cb_s, xpf_s, xpb_s) = refs[i0:]
    B = hf_s.shape[0]

    @pl.when(pl.program_id(0) == 0)
    def _init():
        hf_s[...] = jnp.zeros_like(hf_s)
        cf_s[...] = jnp.zeros_like(cf_s)
        hb_s[...] = jnp.zeros_like(hb_s)
        cb_s[...] = jnp.zeros_like(cb_s)

    def proj(parts, ws, b_ref, ct_ref):
        acc = None
        for k in range(NP):
            if HASC:
                TCp, _, Ik = parts[k].shape
                z = _mmdot(parts[k][...].reshape(TCp * B, Ik), ws[k][...])
                z = z.reshape(TCp, B, 4 * H)
                r = TC // TCp
                if r > 1:
                    z = jnp.broadcast_to(z[:, None, :, :],
                                         (TCp, r, B, 4 * H))
                    z = z.reshape(TC, B, 4 * H)
            else:
                TCp, Ik = parts[k].shape[1], parts[k].shape[2]
                z = _mmdot(parts[k][...].reshape(B * TCp, Ik), ws[k][...])
                z = z.reshape(B, TC, 4 * H)
            acc = z if acc is None else acc + z
        acc = acc + b_ref[0][None, None, :]
        if ct_ref is not None:
            acc = acc + ct_ref[...][None, :, :]
        return acc

    xpf_s[...] = proj(xfs, wihf, bf_ref, ctf_ref if HASC else None)
    xpb_s[...] = proj(xbs, wihb, bb_ref, ctb_ref if HASC else None)

    def xp_at(s, t):
        return s[t] if HASC else s[:, t, :]

    def step(t):
        hf, cf = _cellp(xp_at(xpf_s, t) + _rec(hf_s[...], whf_ref),
                        cf_s[...], H)
        yf_ref[:, pl.ds(t, 1), :] = hf[:, None, :]
        hf_s[...] = hf
        cf_s[...] = cf
        tb = TC - 1 - t
        hb, cb = _cellp(xp_at(xpb_s, tb) + _rec(hb_s[...], whb_ref),
                        cb_s[...], H)
        yb_ref[:, pl.ds(tb, 1), :] = hb[:, None, :]
        hb_s[...] = hb
        cb_s[...] = cb

    def body(t, _):
        for u in range(8):
            step(8 * t + u)
        return 0

    jax.lax.fori_loop(0, TC // 8, body, 0)


def _bilstm_big(xs, Wparts, p, chunks, T, cts=None):
    B = xs[0].shape[1] if cts is not None else xs[0].shape[0]
    H = int(p['fwd'][1].shape[1])
    NP = len(xs)
    NC = chunks
    TC = T // NC
    bfv = _permg(p['fwd'][3] + p['fwd'][2], H).reshape(1, 4 * H)
    bbv = _permg(p['bwd'][3] + p['bwd'][2], H).reshape(1, 4 * H)
    whf = _permg(p['fwd'][1].T, H)
    whb = _permg(p['bwd'][1].T, H)

    def chunk_spec(x, back):
        if cts is not None:
            TCp = TC * x.shape[0] // T
            if back:
                return pl.BlockSpec((TCp, B, int(x.shape[-1])),
                                    lambda i, NC=NC: (NC - 1 - i, 0, 0))
            return pl.BlockSpec((TCp, B, int(x.shape[-1])),
                                lambda i: (i, 0, 0))
        if back:
            return pl.BlockSpec((B, TC, int(x.shape[-1])),
                                lambda i, NC=NC: (0, NC - 1 - i, 0))
        return pl.BlockSpec((B, TC, int(x.shape[-1])),
                            lambda i: (0, i, 0))

    ins = (list(xs) + list(xs) + list(Wparts[0]) + list(Wparts[1])
           + (list(cts) if cts else [])
           + [bfv, bbv, whf, whb])
    in_specs = (
        [chunk_spec(x, False) for x in xs]
        + [chunk_spec(x, True) for x in xs]
        + [pl.BlockSpec((int(w.shape[0]), 4 * H), lambda i: (0, 0))
           for w in list(Wparts[0]) + list(Wparts[1])]
        + ([pl.BlockSpec((B, 4 * H), lambda i: (0, 0))] * 2 if cts else [])
        + [pl.BlockSpec((1, 4 * H), lambda i: (0, 0))] * 2
        + [pl.BlockSpec((H, 4 * H), lambda i: (0, 0))] * 2
    )
    yf, yb = pl.pallas_call(
        functools.partial(_dec_kern, TC=TC, H=H, NP=NP,
                          HASC=cts is not None),
        grid=(NC,),
        in_specs=in_specs,
        out_specs=[
            pl.BlockSpec((B, TC, H), lambda i: (0, i, 0)),
            pl.BlockSpec((B, TC, H), lambda i, NC=NC: (0, NC - 1 - i, 0)),
        ],
        out_shape=[jax.ShapeDtypeStruct((B, T, H), jnp.float32)] * 2,
        scratch_shapes=[pltpu.VMEM((B, H), jnp.float32)] * 4
        + [pltpu.VMEM((TC, B, 4 * H) if cts is not None else
                      (B, TC, 4 * H), jnp.float32)] * 2,
        compiler_params=pltpu.CompilerParams(
            dimension_semantics=("arbitrary",)),
    )(*ins)
    return yf, yb


def kernel(x_f0, x_org, c_trg, params):
    B, T, _ = x_f0.shape
    c = x_f0[:, :, :8]
    f = x_f0[:, :, 8:]
    for i in range(2):
        c = _conv1d(c, params['conv_c'][i]['W'], params['conv_c'][i]['b'])
        f = _conv1d(f, params['conv_f'][i]['W'], params['conv_f'][i]['b'])
    c = _conv1d(c, params['conv_c'][2]['W'], params['conv_c'][2]['b'],
                gn=params['gn_c'], gn_groups=32)
    f = _conv1d(f, params['conv_f'][2]['W'], params['conv_f'][2]['b'],
                gn=params['gn_f'], gn_groups=16)
    r = _conv1d(x_org, params['conv_r']['W'], params['conv_r']['b'],
                gn=params['gn_r'], gn_groups=8)
    cof, cob, offH = _bilstm_multi(
        [c, f, r], [params['lstm_c'], params['lstm_f'], params['lstm_r']])
    def l1_w(dirp):
        Wih = dirp[0]
        H = 512
        WT = _permg(Wih.T, H)
        wf_rows = jnp.zeros((128, 4 * H), jnp.float32)
        wf_rows = wf_rows.at[0:8].set(WT[0:8])
        wf_rows = wf_rows.at[8:40].set(WT[18:50])
        wf_rows = wf_rows.at[40:41].set(WT[16:17])
        wb_rows = jnp.zeros((128, 4 * H), jnp.float32)
        wb_rows = wb_rows.at[0:8].set(WT[8:16])
        wb_rows = wb_rows.at[8:40].set(WT[50:82])
        wb_rows = wb_rows.at[40:41].set(WT[17:18])
        wct = WT[82:164]
        ct = _matmul(c_trg, wct, jnp.zeros((4 * H,), jnp.float32))
        return wf_rows, wb_rows, ct

    l1f_c, l1f_b, ctf = l1_w(params['lstm_d'][0]['fwd'])
    l1b_c, l1b_b, ctb = l1_w(params['lstm_d'][0]['bwd'])
    hs = list(_bilstm_big(
        [cof, cob], ([l1f_c, l1f_b], [l1b_c, l1b_b]),
        params['lstm_d'][0], chunks=6, T=T, cts=(ctf, ctb)))
    for layer in params['lstm_d'][1:]:
        offs = [0, 512, 1024]
        Wp = []
        for dirn in ('fwd', 'bwd'):
            WT = _permg(layer[dirn][0].T, 512)
            Wp.append([WT[offs[k]:offs[k + 1]] for k in range(2)])
        hs = list(_bilstm_big(hs, (Wp[0], Wp[1]), layer, chunks=6, T=T))
    lin = params['linear']
    WT = lin['W'].T
    out = _matmul_multi(
        [h.reshape(B * T, h.shape[-1]) for h in hs],
        [WT[:512], WT[512:]], lin['b'])
    return out.reshape(B, T, lin['W'].shape[0])


# speedup vs baseline: 5.5360x; 1.0030x over previous
import functools

import jax
import jax.numpy as jnp
import numpy as np
from jax.experimental import pallas as pl
from jax.experimental.pallas import tpu as pltpu


def _mmdot(a, b):
    return jax.lax.dot_general(a, b, (((1,), (0,)), ((), ())),
                               preferred_element_type=jnp.float32)


def _conv_kern(x_ref, w_ref, b_ref, o_ref, *, K, T, gn_groups, g_ref=None,
               bt_ref=None, mcg_ref=None, mgc_ref=None):
    acc = jnp.zeros((T, w_ref.shape[2]), jnp.float32)
    for k in range(K):
        acc += _mmdot(x_ref[0, k:k + T, :], w_ref[k])
    acc = jnp.maximum(acc + b_ref[0][None, :], 0.0)
    if gn_groups:
        C = acc.shape[1]
        cs = C // gn_groups
        n = float(cs * T)
        s1 = jnp.sum(acc, axis=0, keepdims=True)
        s2 = jnp.sum(acc * acc, axis=0, keepdims=True)
        g1 = _mmdot(s1, mcg_ref[...]) / n
        g2 = _mmdot(s2, mcg_ref[...]) / n
        rstd = jax.lax.rsqrt(g2 - g1 * g1 + 1e-5)
        mu_c = _mmdot(g1, mgc_ref[...])
        rs_c = _mmdot(rstd, mgc_ref[...])
        a = rs_c * g_ref[0][None, :]
        acc = jnp.maximum(acc * a + (bt_ref[0][None, :] - mu_c * a), 0.0)
    o_ref[0] = acc


def _conv1d(x, W, b, gn=None, gn_groups=0):
    B, T, cin = x.shape
    cout, _, K = W.shape
    pad = (K - 1) // 2
    xp = jnp.pad(x, ((0, 0), (pad, pad), (0, 0)))
    wt = jnp.transpose(W, (2, 1, 0))
    b2 = b.reshape(1, cout)
    ins = [xp, wt, b2]
    in_specs = [
        pl.BlockSpec((1, T + K - 1, cin), lambda i: (i, 0, 0)),
        pl.BlockSpec((K, cin, cout), lambda i: (0, 0, 0)),
        pl.BlockSpec((1, cout), lambda i: (0, 0)),
    ]
    if gn_groups:
        mcg = jnp.repeat(jnp.eye(gn_groups, dtype=jnp.float32),
                         cout // gn_groups, axis=0)
        ins += [gn['g'].reshape(1, cout), gn['b'].reshape(1, cout),
                mcg, mcg.T]
        in_specs += [pl.BlockSpec((1, cout), lambda i: (0, 0))] * 2 + [
            pl.BlockSpec((cout, gn_groups), lambda i: (0, 0)),
            pl.BlockSpec((gn_groups, cout), lambda i: (0, 0)),
        ]
        kern = functools.partial(_conv_kern, K=K, T=T, gn_groups=gn_groups)

        def wrapped(x_r, w_r, b_r, g_r, bt_r, mcg_r, mgc_r, o_r):
            kern(x_r, w_r, b_r, o_r, g_ref=g_r, bt_ref=bt_r,
                 mcg_ref=mcg_r, mgc_ref=mgc_r)
        body = wrapped
    else:
        body = functools.partial(_conv_kern, K=K, T=T, gn_groups=0)
    return pl.pallas_call(
        body,
        grid=(B,),
        in_specs=in_specs,
        out_specs=pl.BlockSpec((1, T, cout), lambda i: (i, 0, 0)),
        out_shape=jax.ShapeDtypeStruct((B, T, cout), jnp.float32),
    )(*ins)


def _mm_kern(*refs, n):
    o_ref, b_ref = refs[-1], refs[-2]
    z = b_ref[0][None, :]
    for k in range(n):
        z = z + _mmdot(refs[k][...], refs[n + k][...])
    o_ref[...] = z


def _matmul_multi(As, Ws, bias):
    M = As[0].shape[0]
    N = Ws[0].shape[1]
    n = len(As)
    bm = M if M <= 768 else 768
    assert M % bm == 0
    bn = N if N <= 2048 else 2048
    assert N % bn == 0
    in_specs = [pl.BlockSpec((bm, int(a.shape[1])),
                             lambda j, i: (i, 0)) for a in As]
    in_specs += [pl.BlockSpec((int(w.shape[0]), bn),
                              lambda j, i: (0, j)) for w in Ws]
    in_specs += [pl.BlockSpec((1, bn), lambda j, i: (0, j))]
    return pl.pallas_call(
        functools.partial(_mm_kern, n=n),
        grid=(N // bn, M // bm),
        in_specs=in_specs,
        out_specs=pl.BlockSpec((bm, bn), lambda j, i: (i, j)),
        out_shape=jax.ShapeDtypeStruct((M, N), jnp.float32),
    )(*As, *Ws, bias.reshape(1, N))


def _matmul(a, w, bias):
    return _matmul_multi([a], [w], bias)


_GSRC = (0, 1, 3, 2)


def _permg(w4, H):
    return jnp.concatenate([w4[..., q * H:(q + 1) * H] for q in _GSRC],
                           axis=-1)


def _cellp(z, c, H):
    s = jax.nn.sigmoid(z[:, :3 * H])
    g = jnp.tanh(z[:, 3 * H:])
    c = s[:, H:2 * H] * c + s[:, :H] * g
    h = s[:, 2 * H:3 * H] * jnp.tanh(c)
    return h, c


def _rec(h, w_ref):
    return jax.lax.dot_general(h, w_ref[...], (((1,), (0,)), ((), ())),
                               preferred_element_type=jnp.float32)


def _bilstm_pack_kern(xf_ref, xb_ref, wf_ref, wb_ref, cof_ref, cob_ref, *,
                      T, HT):
    B = xf_ref.shape[0]
    zero = jnp.zeros((B, HT), jnp.float32)

    def step(t, carry):
        hf, cf, hb, cb = carry
        tb = T - 1 - t
        hf, cf = _cellp(xf_ref[:, t, :] + _rec(hf, wf_ref), cf, HT)
        hb, cb = _cellp(xb_ref[:, tb, :] + _rec(hb, wb_ref), cb, HT)
        return hf, cf, hb, cb

    def body(k, carry):
        for u in range(8):
            carry = step(8 * k + u, carry)
        hf, cf, hb, cb = carry
        cof_ref[pl.ds(k, 1), :, :] = hf[None, :, :]
        cob_ref[pl.ds(T // 8 - 1 - k, 1), :, :] = hb[None, :, :]
        return carry

    jax.lax.fori_loop(0, T // 8, body, (zero, zero, zero, zero))


def _bilstm_multi(xs, ps):
    B, T = xs[0].shape[:2]
    Hs = [int(p['fwd'][1].shape[1]) for p in ps]
    Is = [int(x.shape[-1]) for x in xs]
    HT, IT = 128, sum(Is)
    offH = np.concatenate([[0], np.cumsum(Hs)])
    offI = np.concatenate([[0], np.cumsum(Is)])
    Wall = jnp.zeros((IT, 8 * HT), jnp.float32)
    ball = jnp.zeros((8 * HT,), jnp.float32)
    Whh_d = [jnp.zeros((HT, 4 * HT), jnp.float32) for _ in range(2)]
    for k, p in enumerate(ps):
        H = Hs[k]
        for d, dirn in enumerate(('fwd', 'bwd')):
            Wih, Whh, bih, bhh = p[dirn]
            WT, HhT, bb = Wih.T, Whh.T, bih + bhh
            for q in range(4):
                sq = _GSRC[q]
                col = d * 4 * HT + q * HT + offH[k]
                Wall = Wall.at[offI[k]:offI[k + 1], col:col + H].set(
                    WT[:, sq * H:(sq + 1) * H])
                ball = ball.at[col:col + H].set(bb[sq * H:(sq + 1) * H])
                rcol = q * HT + offH[k]
                Whh_d[d] = Whh_d[d].at[offH[k]:offH[k + 1],
                                       rcol:rcol + H].set(
                    HhT[:, sq * H:(sq + 1) * H])
    xcat = jnp.concatenate(xs, axis=-1).reshape(B * T, IT)
    xp = _matmul(xcat, Wall, ball).reshape(B, T, 8 * HT)
    xf, xb = xp[..., :4 * HT], xp[..., 4 * HT:]
    ins = (xf, xb, Whh_d[0], Whh_d[1])
    cof, cob = pl.pallas_call(
        functools.partial(_bilstm_pack_kern, T=T, HT=HT),
        in_specs=[pl.BlockSpec(a.shape, functools.partial(
                      lambda nd: (0,) * nd, a.ndim)) for a in ins],
        out_specs=[pl.BlockSpec((T // 8, B, HT), lambda: (0, 0, 0))] * 2,
        out_shape=[jax.ShapeDtypeStruct((T // 8, B, HT), jnp.float32)] * 2,
    )(*ins)
    return cof, cob, offH


def _dec_kern(*refs, TC, H, NP, HASC, NC=0):
    xfs = refs[0:NP]
    xbs = refs[NP:2 * NP]
    wihf = refs[2 * NP:3 * NP]
    wihb = refs[3 * NP:4 * NP]
    i0 = 4 * NP
    if HASC:
        ctf_ref, ctb_ref = refs[i0:i0 + 2]
        i0 += 2
    if NC:
        (bf_ref, bb_ref, whf_ref, whb_ref, hwf_ref, hwb_ref, hb_ref,
         out_ref, hf_s, cf_s, hb_s, cb_s, xpf_s, xpb_s,
         yf_ref, yb_ref) = refs[i0:]
    else:
        (bf_ref, bb_ref, whf_ref, whb_ref, yf_ref, yb_ref,
         hf_s, cf_s, hb_s, cb_s, xpf_s, xpb_s) = refs[i0:]
    B = hf_s.shape[0]

    @pl.when(pl.program_id(0) == 0)
    def _init():
        hf_s[...] = jnp.zeros_like(hf_s)
        cf_s[...] = jnp.zeros_like(cf_s)
        hb_s[...] = jnp.zeros_like(hb_s)
        cb_s[...] = jnp.zeros_like(cb_s)

    def proj(parts, ws, b_ref, ct_ref):
        acc = None
        for k in range(NP):
            if HASC:
                TCp, _, Ik = parts[k].shape
                z = _mmdot(parts[k][...].reshape(TCp * B, Ik), ws[k][...])
                z = z.reshape(TCp, B, 4 * H)
                r = TC // TCp
                if r > 1:
                    z = jnp.broadcast_to(z[:, None, :, :],
                                         (TCp, r, B, 4 * H))
                    z = z.reshape(TC, B, 4 * H)
            else:
                TCp, Ik = parts[k].shape[1], parts[k].shape[2]
                z = _mmdot(parts[k][...].reshape(B * TCp, Ik), ws[k][...])
                z = z.reshape(B, TC, 4 * H)
            acc = z if acc is None else acc + z
        acc = acc + b_ref[0][None, None, :]
        if ct_ref is not None:
            acc = acc + ct_ref[...][None, :, :]
        return acc

    xpf_s[...] = proj(xfs, wihf, bf_ref, ctf_ref if HASC else None)
    xpb_s[...] = proj(xbs, wihb, bb_ref, ctb_ref if HASC else None)

    def xp_at(s, t):
        return s[t] if HASC else s[:, t, :]

    def step(t):
        hf, cf = _cellp(xp_at(xpf_s, t) + _rec(hf_s[...], whf_ref),
                        cf_s[...], H)
        yf_ref[:, pl.ds(t, 1), :] = hf[:, None, :]
        hf_s[...] = hf
        cf_s[...] = cf
        tb = TC - 1 - t
        hb, cb = _cellp(xp_at(xpb_s, tb) + _rec(hb_s[...], whb_ref),
                        cb_s[...], H)
        yb_ref[:, pl.ds(tb, 1), :] = hb[:, None, :]
        hb_s[...] = hb
        cb_s[...] = cb

    def body(t, _):
        for u in range(8):
            step(8 * t + u)
        return 0

    jax.lax.fori_loop(0, TC // 8, body, 0)

    if NC:
        i = pl.program_id(0)
        NO = hwf_ref.shape[1]

        @pl.when(i == 0)
        def _initout():
            out_ref[...] = jnp.broadcast_to(hb_ref[0][None, None, :],
                                            out_ref.shape)
        zf = _mmdot(yf_ref[...].reshape(B * TC, H), hwf_ref[...])
        out_ref[:, pl.ds(i * TC, TC), :] += zf.reshape(B, TC, NO)
        zb = _mmdot(yb_ref[...].reshape(B * TC, H), hwb_ref[...])
        out_ref[:, pl.ds((NC - 1 - i) * TC, TC), :] += zb.reshape(B, TC, NO)


def _bilstm_big(xs, Wparts, p, chunks, T, cts=None, head=None):
    B = xs[0].shape[1] if cts is not None else xs[0].shape[0]
    H = int(p['fwd'][1].shape[1])
    NP = len(xs)
    NC = chunks
    TC = T // NC
    bfv = _permg(p['fwd'][3] + p['fwd'][2], H).reshape(1, 4 * H)
    bbv = _permg(p['bwd'][3] + p['bwd'][2], H).reshape(1, 4 * H)
    whf = _permg(p['fwd'][1].T, H)
    whb = _permg(p['bwd'][1].T, H)

    def chunk_spec(x, back):
        if cts is not None:
            TCp = TC * x.shape[0] // T
            if back:
                return pl.BlockSpec((TCp, B, int(x.shape[-1])),
                                    lambda i, NC=NC: (NC - 1 - i, 0, 0))
            return pl.BlockSpec((TCp, B, int(x.shape[-1])),
                                lambda i: (i, 0, 0))
        if back:
            return pl.BlockSpec((B, TC, int(x.shape[-1])),
                                lambda i, NC=NC: (0, NC - 1 - i, 0))
        return pl.BlockSpec((B, TC, int(x.shape[-1])),
                            lambda i: (0, i, 0))

    ins = (list(xs) + list(xs) + list(Wparts[0]) + list(Wparts[1])
           + (list(cts) if cts else [])
           + [bfv, bbv, whf, whb]
           + (list(head) if head else []))
    NO = int(head[0].shape[1]) if head else 0
    in_specs = (
        [chunk_spec(x, False) for x in xs]
        + [chunk_spec(x, True) for x in xs]
        + [pl.BlockSpec((int(w.shape[0]), 4 * H), lambda i: (0, 0))
           for w in list(Wparts[0]) + list(Wparts[1])]
        + ([pl.BlockSpec((B, 4 * H), lambda i: (0, 0))] * 2 if cts else [])
        + [pl.BlockSpec((1, 4 * H), lambda i: (0, 0))] * 2
        + [pl.BlockSpec((H, 4 * H), lambda i: (0, 0))] * 2
        + ([pl.BlockSpec((H, NO), lambda i: (0, 0))] * 2
           + [pl.BlockSpec((1, NO), lambda i: (0, 0))] if head else [])
    )
    scratch = ([pltpu.VMEM((B, H), jnp.float32)] * 4
               + [pltpu.VMEM((TC, B, 4 * H) if cts is not None else
                             (B, TC, 4 * H), jnp.float32)] * 2
               + ([pltpu.VMEM((B, TC, H), jnp.float32)] * 2 if head else []))
    if head:
        out_specs = pl.BlockSpec((B, T, NO), lambda i: (0, 0, 0))
        out_shape = jax.ShapeDtypeStruct((B, T, NO), jnp.float32)
    else:
        out_specs = [
            pl.BlockSpec((B, TC, H), lambda i: (0, i, 0)),
            pl.BlockSpec((B, TC, H), lambda i, NC=NC: (0, NC - 1 - i, 0)),
        ]
        out_shape = [jax.ShapeDtypeStruct((B, T, H), jnp.float32)] * 2
    return pl.pallas_call(
        functools.partial(_dec_kern, TC=TC, H=H, NP=NP,
                          HASC=cts is not None, NC=NC if head else 0),
        grid=(NC,),
        in_specs=in_specs,
        out_specs=out_specs,
        out_shape=out_shape,
        scratch_shapes=scratch,
        compiler_params=pltpu.CompilerParams(
            dimension_semantics=("arbitrary",)),
    )(*ins)


def kernel(x_f0, x_org, c_trg, params):
    B, T, _ = x_f0.shape
    c = x_f0[:, :, :8]
    f = x_f0[:, :, 8:]
    for i in range(2):
        c = _conv1d(c, params['conv_c'][i]['W'], params['conv_c'][i]['b'])
        f = _conv1d(f, params['conv_f'][i]['W'], params['conv_f'][i]['b'])
    c = _conv1d(c, params['conv_c'][2]['W'], params['conv_c'][2]['b'],
                gn=params['gn_c'], gn_groups=32)
    f = _conv1d(f, params['conv_f'][2]['W'], params['conv_f'][2]['b'],
                gn=params['gn_f'], gn_groups=16)
    r = _conv1d(x_org, params['conv_r']['W'], params['conv_r']['b'],
                gn=params['gn_r'], gn_groups=8)
    cof, cob, offH = _bilstm_multi(
        [c, f, r], [params['lstm_c'], params['lstm_f'], params['lstm_r']])
    def l1_w(dirp):
        Wih = dirp[0]
        H = 512
        WT = _permg(Wih.T, H)
        wf_rows = jnp.zeros((128, 4 * H), jnp.float32)
        wf_rows = wf_rows.at[0:8].set(WT[0:8])
        wf_rows = wf_rows.at[8:40].set(WT[18:50])
        wf_rows = wf_rows.at[40:41].set(WT[16:17])
        wb_rows = jnp.zeros((128, 4 * H), jnp.float32)
        wb_rows = wb_rows.at[0:8].set(WT[8:16])
        wb_rows = wb_rows.at[8:40].set(WT[50:82])
        wb_rows = wb_rows.at[40:41].set(WT[17:18])
        wct = WT[82:164]
        ct = _matmul(c_trg, wct, jnp.zeros((4 * H,), jnp.float32))
        return wf_rows, wb_rows, ct

    l1f_c, l1f_b, ctf = l1_w(params['lstm_d'][0]['fwd'])
    l1b_c, l1b_b, ctb = l1_w(params['lstm_d'][0]['bwd'])
    hs = list(_bilstm_big(
        [cof, cob], ([l1f_c, l1f_b], [l1b_c, l1b_b]),
        params['lstm_d'][0], chunks=6, T=T, cts=(ctf, ctb)))
    lin = params['linear']
    NOUT = lin['W'].shape[0]
    LWT = jnp.pad(lin['W'].T, ((0, 0), (0, 128 - NOUT)))
    lbias = jnp.pad(lin['b'], (0, 128 - NOUT)).reshape(1, 128)
    for li, layer in enumerate(params['lstm_d'][1:]):
        offs = [0, 512, 1024]
        Wp = []
        for dirn in ('fwd', 'bwd'):
            WT = _permg(layer[dirn][0].T, 512)
            Wp.append([WT[offs[k]:offs[k + 1]] for k in range(2)])
        head = ((LWT[:512], LWT[512:], lbias)
                if li == len(params['lstm_d']) - 2 else None)
        res = _bilstm_big(hs, (Wp[0], Wp[1]), layer, chunks=6, T=T,
                          head=head)
        if head is None:
            hs = list(res)
    return res[:, :, :NOUT]
